# baseline (speedup 1.0000x reference)
import functools

import jax
import jax.numpy as jnp
from jax.experimental import pallas as pl
from jax.experimental.pallas import tpu as pltpu

_B, _C, _N = 64, 32, 4096
_TN = 2048


@functools.lru_cache(maxsize=1)
def _gumbel_const():
    eps = 1e-20
    u = jax.random.uniform(jax.random.key(42), (_B, _C, _N), dtype=jnp.float32)
    return -jnp.log(-jnp.log(u + eps) + eps)


def _body(l_ref, g_ref, o_ref):
    l = l_ref[0]
    g = g_ref[0]
    m = jnp.max(l, axis=0, keepdims=True)
    e = jnp.exp(l - m)
    p = e / jnp.sum(e, axis=0, keepdims=True)
    z = jnp.log(p) + g
    m2 = jnp.max(z, axis=0, keepdims=True)
    u = jnp.exp(z - m2)
    y = u / jnp.sum(u, axis=0, keepdims=True)
    hit = y[0:1, :] >= jnp.max(y, axis=0, keepdims=True)
    o_ref[0] = hit.astype(jnp.float32)


def kernel(logits):
    g = _gumbel_const()
    grid = (_B, _N // _TN)
    return pl.pallas_call(
        _body,
        grid=grid,
        in_specs=[
            pl.BlockSpec((1, _C, _TN), lambda b, n: (b, 0, n)),
            pl.BlockSpec((1, _C, _TN), lambda b, n: (b, 0, n)),
        ],
        out_specs=pl.BlockSpec((1, 1, _TN), lambda b, n: (b, 0, n)),
        out_shape=jax.ShapeDtypeStruct((_B, 1, _N), jnp.float32),
        compiler_params=pltpu.CompilerParams(
            dimension_semantics=("parallel", "parallel"),
        ),
    )(logits, g)


# speedup vs baseline: 1.0248x; 1.0248x over previous
import functools

import jax
import jax.numpy as jnp
from jax.experimental import pallas as pl
from jax.experimental.pallas import tpu as pltpu

_B, _C, _N = 64, 32, 4096
_TN = 2048


@functools.lru_cache(maxsize=1)
def _gumbel_const():
    eps = 1e-20
    u = jax.random.uniform(jax.random.key(42), (_B, _C, _N), dtype=jnp.float32)
    return -jnp.log(-jnp.log(u + eps) + eps)


def _body(l_ref, g_ref, o_ref):
    l = l_ref[0]
    g = g_ref[0]
    m = jnp.max(l, axis=0, keepdims=True)
    e = jnp.exp(l - m)
    p = e / jnp.sum(e, axis=0, keepdims=True)
    z = jnp.log(p) + g
    hit = z[0:1, :] >= jnp.max(z, axis=0, keepdims=True)
    o_ref[0] = hit.astype(jnp.float32)


def kernel(logits):
    g = _gumbel_const()
    grid = (_B, _N // _TN)
    return pl.pallas_call(
        _body,
        grid=grid,
        in_specs=[
            pl.BlockSpec((1, _C, _TN), lambda b, n: (b, 0, n)),
            pl.BlockSpec((1, _C, _TN), lambda b, n: (b, 0, n)),
        ],
        out_specs=pl.BlockSpec((1, 1, _TN), lambda b, n: (b, 0, n)),
        out_shape=jax.ShapeDtypeStruct((_B, 1, _N), jnp.float32),
        compiler_params=pltpu.CompilerParams(
            dimension_semantics=("parallel", "parallel"),
        ),
    )(logits, g)


# speedup vs baseline: 1.1877x; 1.1590x over previous
import functools

import jax
import jax.numpy as jnp
from jax.experimental import pallas as pl
from jax.experimental.pallas import tpu as pltpu

_B, _C, _N = 64, 32, 4096
_TN = 4096


@functools.lru_cache(maxsize=1)
def _gumbel_const():
    eps = 1e-20
    u = jax.random.uniform(jax.random.key(42), (_B, _C, _N), dtype=jnp.float32)
    return -jnp.log(-jnp.log(u + eps) + eps)


def _body(l_ref, g_ref, o_ref):
    l = l_ref[0]
    g = g_ref[0]
    m = jnp.max(l, axis=0, keepdims=True)
    e = jnp.exp(l - m)
    p = e / jnp.sum(e, axis=0, keepdims=True)
    z = jnp.log(p) + g
    hit = z[0:1, :] >= jnp.max(z, axis=0, keepdims=True)
    o_ref[0] = hit.astype(jnp.float32)


def kernel(logits):
    g = _gumbel_const()
    grid = (_B, _N // _TN)
    return pl.pallas_call(
        _body,
        grid=grid,
        in_specs=[
            pl.BlockSpec((1, _C, _TN), lambda b, n: (b, 0, n)),
            pl.BlockSpec((1, _C, _TN), lambda b, n: (b, 0, n)),
        ],
        out_specs=pl.BlockSpec((1, 1, _TN), lambda b, n: (b, 0, n)),
        out_shape=jax.ShapeDtypeStruct((_B, 1, _N), jnp.float32),
        compiler_params=pltpu.CompilerParams(
            dimension_semantics=("parallel", "parallel"),
        ),
    )(logits, g)
